# baseline (speedup 1.0000x reference)
import jax
import jax.numpy as jnp
from jax.experimental import pallas as pl
from jax.experimental.pallas import tpu as pltpu

NUM_EXPERTS = 8
TOP_K = 2
HIDDEN = 1024
INTER = 2816
TOKENS = 2048

PAIRS = TOKENS * TOP_K
BN = 256
NB = INTER // BN
RC = 256
GU = 8
CU = 8
ROWS = PAIRS + 64 + RC


def _gather_groups(x_s, hs_ref, tok_ref, start, g0, g1):
    def gbody(t, _):
        base = start + t * GU
        for u in range(GU):
            tok = tok_ref[base + u]
            x_s[pl.ds(base + u, 1), :] = hs_ref[pl.ds(tok, 1), :]
        return 0
    jax.lax.fori_loop(g0, g1, gbody, 0)


def _combine_groups(out_ref, acc_s, tord_ref, pos_ref, w01_ref, rstart, m0, m1):
    def cbody(t, _):
        base = rstart + t * CU
        for u in range(CU):
            tt = tord_ref[base + u]
            p0 = pos_ref[2 * tt]
            p1 = pos_ref[2 * tt + 1]
            w0 = w01_ref[2 * tt]
            w1 = w01_ref[2 * tt + 1]
            out_ref[pl.ds(tt, 1), :] = (
                w0 * acc_s[pl.ds(p0, 1), :] + w1 * acc_s[pl.ds(p1, 1), :])
        return 0
    jax.lax.fori_loop(m0, m1, cbody, 0)


def _moe_body(counts_ref, starts_ref, tok_ref, pos_ref, w01_ref,
              tord_ref, rcounts_ref, rstarts_ref,
              hs_ref, wg_ref, wu_ref, wd_ref, out_ref,
              x_s, acc_s):
    e = pl.program_id(0)
    n = pl.program_id(1)
    cnt = counts_ref[e]
    start = pl.multiple_of(starts_ref[e], 8)
    nch = (cnt + RC - 1) // RC

    @pl.when((e == 0) & (n == 0))
    def _gather0():
        _gather_groups(x_s, hs_ref, tok_ref, start, 0,
                       (cnt + GU - 1) // GU)

    wg = wg_ref[0].astype(jnp.bfloat16)
    wu = wu_ref[0].astype(jnp.bfloat16)
    wd = wd_ref[0].astype(jnp.bfloat16)

    def chunk(j, add):
        r0 = start + j * RC
        xj = x_s[pl.ds(r0, RC), :].astype(jnp.bfloat16)
        gate = jnp.dot(xj, wg, preferred_element_type=jnp.float32)
        up = jnp.dot(xj, wu, preferred_element_type=jnp.float32)
        h = (gate * jax.nn.sigmoid(gate) * up).astype(jnp.bfloat16)
        y = jnp.dot(h, wd, preferred_element_type=jnp.float32)
        if add:
            acc_s[pl.ds(r0, RC), :] += y
        else:
            acc_s[pl.ds(r0, RC), :] = y
        return 0

    @pl.when(n == 0)
    def _first():
        jax.lax.fori_loop(0, nch, lambda j, c: chunk(j, False), 0)

    @pl.when(n != 0)
    def _rest():
        jax.lax.fori_loop(0, nch, lambda j, c: chunk(j, True), 0)

    @pl.when((n != 0) & (e < NUM_EXPERTS - 1))
    def _gather_next():
        en = e + 1
        cnt_n = counts_ref[en]
        start_n = pl.multiple_of(starts_ref[en], 8)
        groups = (cnt_n + GU - 1) // GU
        gper = (groups + NB - 2) // (NB - 1)
        g0 = (n - 1) * gper
        g1 = jnp.minimum(g0 + gper, groups)
        _gather_groups(x_s, hs_ref, tok_ref, start_n, g0, g1)

    @pl.when((n != 0) & (e > 0))
    def _combine_ready():
        ep = e - 1
        m = rcounts_ref[ep]
        rstart = rstarts_ref[ep]
        groups = (m + CU - 1) // CU
        mper = (groups + NB - 2) // (NB - 1)
        m0 = (n - 1) * mper
        m1 = jnp.minimum(m0 + mper, groups)
        _combine_groups(out_ref, acc_s, tord_ref, pos_ref, w01_ref,
                        rstart, m0, m1)

    @pl.when((e == NUM_EXPERTS - 1) & (n == NB - 1))
    def _combine_last():
        ep = NUM_EXPERTS - 1
        m = rcounts_ref[ep]
        rstart = rstarts_ref[ep]
        _combine_groups(out_ref, acc_s, tord_ref, pos_ref, w01_ref,
                        rstart, 0, (m + CU - 1) // CU)


@jax.jit
def kernel(hidden_states, routing_weights, selected_experts, gate_up_proj, down_proj):
    flat_e = selected_experts.reshape(-1)
    order = jnp.argsort(flat_e)
    sorted_e = flat_e[order]
    counts = jnp.bincount(flat_e, length=NUM_EXPERTS).astype(jnp.int32)
    starts = (jnp.cumsum(counts) - counts).astype(jnp.int32)
    p = jnp.zeros((), jnp.int32)
    ps = []
    for ee in range(NUM_EXPERTS):
        ps.append(p)
        p = ((p + counts[ee]) + 7) // 8 * 8
    pad_starts = jnp.stack(ps).astype(jnp.int32)
    dest = pad_starts[sorted_e] + (
        jnp.arange(PAIRS, dtype=jnp.int32) - starts[sorted_e])
    tok_pad = jnp.zeros((ROWS,), jnp.int32).at[dest].set(
        (order // TOP_K).astype(jnp.int32))
    pos = jnp.zeros((PAIRS,), jnp.int32).at[order].set(dest)
    w01 = routing_weights.reshape(-1)

    ready_e = jnp.max(selected_experts, axis=1)
    tord = jnp.argsort(ready_e).astype(jnp.int32)
    tord = jnp.concatenate([tord, jnp.zeros((CU,), jnp.int32)])
    rcounts = jnp.bincount(ready_e, length=NUM_EXPERTS).astype(jnp.int32)
    rstarts = (jnp.cumsum(rcounts) - rcounts).astype(jnp.int32)

    grid_spec = pltpu.PrefetchScalarGridSpec(
        num_scalar_prefetch=8,
        grid=(NUM_EXPERTS, NB),
        in_specs=[
            pl.BlockSpec((TOKENS, HIDDEN), lambda e, n, *_: (0, 0)),
            pl.BlockSpec((1, HIDDEN, BN), lambda e, n, *_: (e, 0, n)),
            pl.BlockSpec((1, HIDDEN, BN), lambda e, n, *_: (e, 0, n + NB)),
            pl.BlockSpec((1, BN, HIDDEN), lambda e, n, *_: (e, n, 0)),
        ],
        out_specs=pl.BlockSpec((TOKENS, HIDDEN), lambda e, n, *_: (0, 0)),
        scratch_shapes=[
            pltpu.VMEM((ROWS, HIDDEN), jnp.float32),
            pltpu.VMEM((ROWS, HIDDEN), jnp.float32),
        ],
    )

    out = pl.pallas_call(
        _moe_body,
        grid_spec=grid_spec,
        out_shape=jax.ShapeDtypeStruct((TOKENS, HIDDEN), jnp.float32),
        compiler_params=pltpu.CompilerParams(
            dimension_semantics=("arbitrary", "arbitrary"),
        ),
    )(counts, pad_starts, tok_pad, pos, w01, tord, rcounts, rstarts,
      hidden_states, gate_up_proj, gate_up_proj, down_proj)
    return out


# speedup vs baseline: 1.0697x; 1.0697x over previous
import jax
import jax.numpy as jnp
from jax.experimental import pallas as pl
from jax.experimental.pallas import tpu as pltpu

NUM_EXPERTS = 8
TOP_K = 2
HIDDEN = 1024
INTER = 2816
TOKENS = 2048

PAIRS = TOKENS * TOP_K
BN = 256
NB = INTER // BN
RC = 256
GU = 16
CU = 8
ROWS = PAIRS + 128 + RC


def _moe_body(counts_ref, starts_ref, tok_ref, pos_ref, w01_ref,
              hs_ref, wg_ref, wu_ref, wd_ref, out_ref,
              x_s, acc_s):
    e = pl.program_id(0)
    n = pl.program_id(1)
    cnt = counts_ref[e]
    start = pl.multiple_of(starts_ref[e], GU)
    nch = (cnt + RC - 1) // RC

    @pl.when(n == 0)
    def _gather():
        def gbody(t, _):
            base = start + t * GU
            rows = [hs_ref[pl.ds(tok_ref[base + u], 1), :] for u in range(GU)]
            x_s[pl.ds(base, GU), :] = jnp.concatenate(
                rows, axis=0).astype(jnp.bfloat16)
            return 0
        jax.lax.fori_loop(0, (cnt + GU - 1) // GU, gbody, 0)

    wg = wg_ref[0].astype(jnp.bfloat16)
    wu = wu_ref[0].astype(jnp.bfloat16)
    wd = wd_ref[0].astype(jnp.bfloat16)

    def chunk(j, add):
        r0 = start + j * RC
        xj = x_s[pl.ds(r0, RC), :]
        gate = jnp.dot(xj, wg, preferred_element_type=jnp.float32)
        up = jnp.dot(xj, wu, preferred_element_type=jnp.float32)
        h = (gate * jax.nn.sigmoid(gate) * up).astype(jnp.bfloat16)
        y = jnp.dot(h, wd, preferred_element_type=jnp.float32)
        if add:
            acc_s[pl.ds(r0, RC), :] += y
        else:
            acc_s[pl.ds(r0, RC), :] = y
        return 0

    @pl.when(n == 0)
    def _first():
        jax.lax.fori_loop(0, nch, lambda j, c: chunk(j, False), 0)

    @pl.when(n != 0)
    def _rest():
        jax.lax.fori_loop(0, nch, lambda j, c: chunk(j, True), 0)

    @pl.when((e == NUM_EXPERTS - 1) & (n == NB - 1))
    def _combine():
        def cbody(t, _):
            base = t * CU
            for u in range(CU):
                row = base + u
                p0 = pos_ref[2 * row]
                p1 = pos_ref[2 * row + 1]
                w0 = w01_ref[2 * row]
                w1 = w01_ref[2 * row + 1]
                out_ref[pl.ds(row, 1), :] = (
                    w0 * acc_s[pl.ds(p0, 1), :] + w1 * acc_s[pl.ds(p1, 1), :])
            return 0
        jax.lax.fori_loop(0, TOKENS // CU, cbody, 0)


@jax.jit
def kernel(hidden_states, routing_weights, selected_experts, gate_up_proj, down_proj):
    flat_e = selected_experts.reshape(-1)
    order = jnp.argsort(flat_e)
    sorted_e = flat_e[order]
    counts = jnp.bincount(flat_e, length=NUM_EXPERTS).astype(jnp.int32)
    starts = (jnp.cumsum(counts) - counts).astype(jnp.int32)
    p = jnp.zeros((), jnp.int32)
    ps = []
    for ee in range(NUM_EXPERTS):
        ps.append(p)
        p = ((p + counts[ee]) + GU - 1) // GU * GU
    pad_starts = jnp.stack(ps).astype(jnp.int32)
    dest = pad_starts[sorted_e] + (
        jnp.arange(PAIRS, dtype=jnp.int32) - starts[sorted_e])
    tok_pad = jnp.zeros((ROWS,), jnp.int32).at[dest].set(
        (order // TOP_K).astype(jnp.int32))
    pos = jnp.zeros((PAIRS,), jnp.int32).at[order].set(dest)
    w01 = routing_weights.reshape(-1)

    grid_spec = pltpu.PrefetchScalarGridSpec(
        num_scalar_prefetch=5,
        grid=(NUM_EXPERTS, NB),
        in_specs=[
            pl.BlockSpec((TOKENS, HIDDEN), lambda e, n, *_: (0, 0)),
            pl.BlockSpec((1, HIDDEN, BN), lambda e, n, *_: (e, 0, n)),
            pl.BlockSpec((1, HIDDEN, BN), lambda e, n, *_: (e, 0, n + NB)),
            pl.BlockSpec((1, BN, HIDDEN), lambda e, n, *_: (e, n, 0)),
        ],
        out_specs=pl.BlockSpec((TOKENS, HIDDEN), lambda e, n, *_: (0, 0)),
        scratch_shapes=[
            pltpu.VMEM((ROWS, HIDDEN), jnp.bfloat16),
            pltpu.VMEM((ROWS, HIDDEN), jnp.float32),
        ],
    )

    out = pl.pallas_call(
        _moe_body,
        grid_spec=grid_spec,
        out_shape=jax.ShapeDtypeStruct((TOKENS, HIDDEN), jnp.float32),
        compiler_params=pltpu.CompilerParams(
            dimension_semantics=("arbitrary", "arbitrary"),
        ),
    )(counts, pad_starts, tok_pad, pos, w01,
      hidden_states, gate_up_proj, gate_up_proj, down_proj)
    return out
